# Initial kernel scaffold; baseline (speedup 1.0000x reference)
import jax
import jax.numpy as jnp
from jax.experimental import pallas as pl


def kernel(h_src, h_dst, edge_index):
    raise NotImplementedError("write your pallas kernel here")


# speedup vs baseline: 1.8073x; 1.8073x over previous
import functools

import jax
import jax.numpy as jnp
from jax import lax
from jax.experimental import pallas as pl
from jax.experimental.pallas import tpu as pltpu
from jax.experimental.pallas import tpu_sc as plsc

def _lane_shuffle(x, idx):
    dnums = lax.GatherDimensionNumbers(
        offset_dims=(), collapsed_slice_dims=(0,), start_index_map=(0,))
    return lax.gather(x, idx[:, None], dnums, slice_sizes=(1,),
                      mode=lax.GatherScatterMode.PROMISE_IN_BOUNDS)


N_NODES_ = 10000
N_EDGES_ = 320000
D_ = 128

NC_ = 2
NS_ = 16
NW_ = NC_ * NS_
C_ = 128
E_PAD_ = 327680
NCHUNK_ = E_PAD_ // (NW_ * C_)


def _sc_kernel(hs, hd, srcr, dstr, out,
               idx_s, idx_d, rows_s, rows_d, score_all, sem_s, sem_d):
    wid = lax.axis_index("s") * NC_ + lax.axis_index("c")
    pltpu.sync_copy(srcr.at[wid], idx_s)
    pltpu.sync_copy(dstr.at[wid], idx_d)

    lane = lax.iota(jnp.int32, 16)
    perms = [(lane + s) & 15 for s in (8, 4, 2, 1)]

    @pl.loop(0, NCHUNK_)
    def _chunk(k):
        cp_s = pltpu.async_copy(hs.at[idx_s.at[k]], rows_s, sem_s)
        cp_d = pltpu.async_copy(hd.at[idx_d.at[k]], rows_d, sem_d)
        cp_s.wait()
        cp_d.wait()

        @pl.loop(0, C_ // 16)
        def _group(g):
            score = jnp.zeros((16,), jnp.float32)
            for u in range(16):
                e = g * 16 + u
                p = [rows_s[e, 16 * f:16 * (f + 1)] *
                     rows_d[e, 16 * f:16 * (f + 1)] for f in range(8)]
                acc = ((p[0] + p[1]) + (p[2] + p[3])) + \
                      ((p[4] + p[5]) + (p[6] + p[7]))
                for pm in perms:
                    acc = acc + _lane_shuffle(acc, pm)
                score = jnp.where(lane == u, acc, score)
            score_all[k, pl.ds(g * 16, 16)] = score

    pltpu.sync_copy(score_all, out.at[wid])


@jax.jit
def kernel(h_src, h_dst, edge_index):
    ei = edge_index.astype(jnp.int32)
    pad = E_PAD_ - N_EDGES_
    src = jnp.pad(ei[0], (0, pad)).reshape(NW_, NCHUNK_, C_)
    dst = jnp.pad(ei[1], (0, pad)).reshape(NW_, NCHUNK_, C_)

    mesh = plsc.VectorSubcoreMesh(core_axis_name="c", subcore_axis_name="s")
    sck = functools.partial(
        pl.kernel,
        out_type=jax.ShapeDtypeStruct((NW_, NCHUNK_, C_), jnp.float32),
        mesh=mesh,
        scratch_types=[
            pltpu.VMEM((NCHUNK_, C_), jnp.int32),
            pltpu.VMEM((NCHUNK_, C_), jnp.int32),
            pltpu.VMEM((C_, D_), jnp.float32),
            pltpu.VMEM((C_, D_), jnp.float32),
            pltpu.VMEM((NCHUNK_, C_), jnp.float32),
            pltpu.SemaphoreType.DMA,
            pltpu.SemaphoreType.DMA,
        ],
    )(_sc_kernel)
    out = sck(h_src, h_dst, src, dst)
    return out.reshape(-1)[:N_EDGES_].reshape(N_EDGES_, 1)


# speedup vs baseline: 2.9239x; 1.6178x over previous
import functools

import jax
import jax.numpy as jnp
from jax import lax
from jax.experimental import pallas as pl
from jax.experimental.pallas import tpu as pltpu
from jax.experimental.pallas import tpu_sc as plsc


def _lane_shuffle(x, idx):
    dnums = lax.GatherDimensionNumbers(
        offset_dims=(), collapsed_slice_dims=(0,), start_index_map=(0,))
    return lax.gather(x, idx[:, None], dnums, slice_sizes=(1,),
                      mode=lax.GatherScatterMode.PROMISE_IN_BOUNDS)


N_NODES_ = 10000
N_EDGES_ = 320000
D_ = 128

NC_ = 2
NS_ = 16
NW_ = NC_ * NS_
C_ = 64
NBUF_ = 4
E_PAD_ = 327680
NCHUNK_ = E_PAD_ // (NW_ * C_)


def _sc_kernel(hs, hd, srcr, dstr, out,
               idx_s, idx_d, rows_s, rows_d, score_all, sem_s, sem_d):
    wid = lax.axis_index("s") * NC_ + lax.axis_index("c")
    pltpu.sync_copy(srcr.at[wid], idx_s)
    pltpu.sync_copy(dstr.at[wid], idx_d)

    lane = lax.iota(jnp.int32, 16)
    perms = [(lane + s) & 15 for s in (8, 4, 2, 1)]

    def _issue(k, slot):
        pltpu.async_copy(hs.at[idx_s.at[k]], rows_s.at[slot], sem_s.at[slot])
        pltpu.async_copy(hd.at[idx_d.at[k]], rows_d.at[slot], sem_d.at[slot])

    for s0 in range(NBUF_):
        _issue(s0, s0)

    @pl.loop(0, NCHUNK_)
    def _chunk(k):
        slot = lax.rem(k, NBUF_)
        pltpu.make_async_copy(hs.at[idx_s.at[k]], rows_s.at[slot],
                              sem_s.at[slot]).wait()
        pltpu.make_async_copy(hd.at[idx_d.at[k]], rows_d.at[slot],
                              sem_d.at[slot]).wait()

        @pl.loop(0, C_ // 16)
        def _group(g):
            @pl.loop(0, 16, init_carry=jnp.zeros((16,), jnp.float32))
            def _edge(u, score):
                e = g * 16 + u
                p = [rows_s[slot, e, 16 * f:16 * (f + 1)] *
                     rows_d[slot, e, 16 * f:16 * (f + 1)] for f in range(8)]
                acc = ((p[0] + p[1]) + (p[2] + p[3])) + \
                      ((p[4] + p[5]) + (p[6] + p[7]))
                for pm in perms:
                    acc = acc + _lane_shuffle(acc, pm)
                return jnp.where(lane == u, acc, score)
            score_all[k, pl.ds(g * 16, 16)] = _edge

        @pl.when(k + NBUF_ < NCHUNK_)
        def _refill():
            _issue(k + NBUF_, slot)

    pltpu.sync_copy(score_all, out.at[wid])


@jax.jit
def kernel(h_src, h_dst, edge_index):
    ei = edge_index.astype(jnp.int32)
    pad = E_PAD_ - N_EDGES_
    src = jnp.pad(ei[0], (0, pad)).reshape(NW_, NCHUNK_, C_)
    dst = jnp.pad(ei[1], (0, pad)).reshape(NW_, NCHUNK_, C_)

    mesh = plsc.VectorSubcoreMesh(core_axis_name="c", subcore_axis_name="s")
    sck = functools.partial(
        pl.kernel,
        out_type=jax.ShapeDtypeStruct((NW_, NCHUNK_, C_), jnp.float32),
        mesh=mesh,
        scratch_types=[
            pltpu.VMEM((NCHUNK_, C_), jnp.int32),
            pltpu.VMEM((NCHUNK_, C_), jnp.int32),
            pltpu.VMEM((NBUF_, C_, D_), jnp.float32),
            pltpu.VMEM((NBUF_, C_, D_), jnp.float32),
            pltpu.VMEM((NCHUNK_, C_), jnp.float32),
            pltpu.SemaphoreType.DMA((NBUF_,)),
            pltpu.SemaphoreType.DMA((NBUF_,)),
        ],
    )(_sc_kernel)
    out = sck(h_src, h_dst, src, dst)
    return out.reshape(-1)[:N_EDGES_].reshape(N_EDGES_, 1)
